# baseline (speedup 1.0000x reference)
import functools

import jax
import jax.numpy as jnp
from jax import lax
from jax.experimental import pallas as pl
from jax.experimental.pallas import tpu as pltpu
from jax.experimental.pallas import tpu_sc as plsc

NC = 2
NS = 16
L = 16
CHUNK = 128


def kernel(u_id, i_id, user_emb, user_bias, item_emb, item_bias, mean):
    B = u_id.shape[0]
    D = user_emb.shape[1]
    nw = NC * NS
    n_per_w = B // nw
    assert B % (nw * L) == 0 and n_per_w % CHUNK == 0
    n_chunks = n_per_w // CHUNK

    mesh = plsc.VectorSubcoreMesh(core_axis_name="c", subcore_axis_name="s")

    @functools.partial(
        pl.kernel,
        out_type=jax.ShapeDtypeStruct((B,), jnp.float32),
        mesh=mesh,
        compiler_params=pltpu.CompilerParams(needs_layout_passes=False),
        scratch_types=[
            pltpu.VMEM((CHUNK,), jnp.int32),
            pltpu.VMEM((CHUNK,), jnp.int32),
            pltpu.VMEM((CHUNK, D), jnp.float32),
            pltpu.VMEM((CHUNK, D), jnp.float32),
            pltpu.VMEM((CHUNK,), jnp.float32),
            pltpu.VMEM((CHUNK,), jnp.float32),
            pltpu.VMEM((L,), jnp.float32),
            pltpu.VMEM((B // (NC * NS),), jnp.float32),
            pltpu.SemaphoreType.DMA,
        ],
    )
    def mf_kernel(uid_h, iid_h, uemb_h, ubias_h, iemb_h, ibias_h, mean_h,
                  out_h, uidx_v, iidx_v, urows_v, irows_v, ub_v, ib_v,
                  mean_v, out_v, sem):
        c = lax.axis_index("c")
        s = lax.axis_index("s")
        wid = s * NC + c
        base = wid * n_per_w

        pltpu.sync_copy(mean_h, mean_v.at[pl.ds(0, 1)])
        mean_vec = plsc.load_gather(mean_v, [jnp.zeros((L,), jnp.int32)])

        for k in range(n_chunks):
            cb = base + k * CHUNK
            pltpu.sync_copy(uid_h.at[pl.ds(cb, CHUNK)], uidx_v)
            pltpu.sync_copy(iid_h.at[pl.ds(cb, CHUNK)], iidx_v)
            pltpu.async_copy(uemb_h.at[uidx_v], urows_v, sem)
            pltpu.async_copy(iemb_h.at[iidx_v], irows_v, sem)
            pltpu.async_copy(ubias_h.at[uidx_v], ub_v, sem)
            pltpu.async_copy(ibias_h.at[iidx_v], ib_v, sem)
            pltpu.make_async_copy(uemb_h.at[uidx_v], urows_v, sem).wait()
            pltpu.make_async_copy(iemb_h.at[iidx_v], irows_v, sem).wait()
            pltpu.make_async_copy(ubias_h.at[uidx_v], ub_v, sem).wait()
            pltpu.make_async_copy(ibias_h.at[iidx_v], ib_v, sem).wait()

            for g in range(CHUNK // L):
                rows = jnp.full((L,), g * L, jnp.int32) + lax.iota(jnp.int32, L)
                acc = ub_v[pl.ds(g * L, L)] + ib_v[pl.ds(g * L, L)] + mean_vec

                def dbody(d, acc):
                    cols = jnp.full((L,), 0, jnp.int32) + d
                    u = plsc.load_gather(urows_v, [rows, cols])
                    it = plsc.load_gather(irows_v, [rows, cols])
                    return acc + u * it

                acc = lax.fori_loop(0, D, dbody, acc, unroll=8)
                out_v[pl.ds(k * CHUNK + g * L, L)] = acc

        pltpu.sync_copy(out_v, out_h.at[pl.ds(base, n_per_w)])

    return mf_kernel(u_id, i_id, user_emb, user_bias.reshape(-1),
                     item_emb, item_bias.reshape(-1), mean)


# speedup vs baseline: 2.4388x; 2.4388x over previous
import functools

import jax
import jax.numpy as jnp
from jax import lax
from jax.experimental import pallas as pl
from jax.experimental.pallas import tpu as pltpu
from jax.experimental.pallas import tpu_sc as plsc

NC = 2
NS = 16
L = 16
CHUNK = 128


def kernel(u_id, i_id, user_emb, user_bias, item_emb, item_bias, mean):
    B = u_id.shape[0]
    D = user_emb.shape[1]
    nw = NC * NS
    n_per_w = B // nw
    assert B % (nw * L) == 0 and n_per_w % CHUNK == 0
    n_chunks = n_per_w // CHUNK

    mesh = plsc.VectorSubcoreMesh(core_axis_name="c", subcore_axis_name="s")

    @functools.partial(
        pl.kernel,
        out_type=jax.ShapeDtypeStruct((B,), jnp.float32),
        mesh=mesh,
        compiler_params=pltpu.CompilerParams(needs_layout_passes=False),
        scratch_types=[
            pltpu.VMEM((CHUNK,), jnp.int32),
            pltpu.VMEM((CHUNK,), jnp.int32),
            pltpu.VMEM((CHUNK,), jnp.int32),
            pltpu.VMEM((CHUNK,), jnp.int32),
            pltpu.VMEM((CHUNK, D), jnp.float32),
            pltpu.VMEM((CHUNK, D), jnp.float32),
            pltpu.VMEM((CHUNK, D), jnp.float32),
            pltpu.VMEM((CHUNK, D), jnp.float32),
            pltpu.VMEM((CHUNK,), jnp.float32),
            pltpu.VMEM((CHUNK,), jnp.float32),
            pltpu.VMEM((CHUNK,), jnp.float32),
            pltpu.VMEM((CHUNK,), jnp.float32),
            pltpu.VMEM((L,), jnp.float32),
            pltpu.VMEM((L, L), jnp.float32),
            pltpu.VMEM((B // (NC * NS),), jnp.float32),
            pltpu.SemaphoreType.DMA,
            pltpu.SemaphoreType.DMA,
        ],
    )
    def mf_kernel(uid_h, iid_h, uemb_h, ubias_h, iemb_h, ibias_h, mean_h,
                  out_h, uidx0_v, uidx1_v, iidx0_v, iidx1_v, urows0_v,
                  urows1_v, irows0_v, irows1_v, ub0_v, ub1_v, ib0_v, ib1_v,
                  mean_v, tmp_v, out_v, sem0, sem1):
        c = lax.axis_index("c")
        s = lax.axis_index("s")
        wid = s * NC + c
        base = wid * n_per_w

        uidx = [uidx0_v, uidx1_v]
        iidx = [iidx0_v, iidx1_v]
        urows = [urows0_v, urows1_v]
        irows = [irows0_v, irows1_v]
        ub = [ub0_v, ub1_v]
        ib = [ib0_v, ib1_v]
        sems = [sem0, sem1]

        pltpu.sync_copy(mean_h, mean_v.at[pl.ds(0, 1)])
        mean_vec = plsc.load_gather(mean_v, [jnp.zeros((L,), jnp.int32)])
        iota = lax.iota(jnp.int32, L)

        def fire(k, sl):
            cb = base + k * CHUNK
            pltpu.sync_copy(uid_h.at[pl.ds(cb, CHUNK)], uidx[sl])
            pltpu.sync_copy(iid_h.at[pl.ds(cb, CHUNK)], iidx[sl])
            pltpu.async_copy(uemb_h.at[uidx[sl]], urows[sl], sems[sl])
            pltpu.async_copy(iemb_h.at[iidx[sl]], irows[sl], sems[sl])
            pltpu.async_copy(ubias_h.at[uidx[sl]], ub[sl], sems[sl])
            pltpu.async_copy(ibias_h.at[iidx[sl]], ib[sl], sems[sl])

        def drain(k, sl):
            pltpu.make_async_copy(uemb_h.at[uidx[sl]], urows[sl], sems[sl]).wait()
            pltpu.make_async_copy(iemb_h.at[iidx[sl]], irows[sl], sems[sl]).wait()
            pltpu.make_async_copy(ubias_h.at[uidx[sl]], ub[sl], sems[sl]).wait()
            pltpu.make_async_copy(ibias_h.at[iidx[sl]], ib[sl], sems[sl]).wait()

        fire(0, 0)
        fire(1, 1)
        for k in range(n_chunks):
            sl = k % 2
            drain(k, sl)
            urows_v, irows_v, ub_v, ib_v = urows[sl], irows[sl], ub[sl], ib[sl]

            def group_body(g, _):
                gbase = g * L
                for j in range(L):
                    b = gbase + j
                    p0 = urows_v[b, pl.ds(0, L)] * irows_v[b, pl.ds(0, L)]
                    p1 = urows_v[b, pl.ds(L, L)] * irows_v[b, pl.ds(L, L)]
                    for slc in range(2, D // L, 2):
                        p0 = p0 + (urows_v[b, pl.ds(slc * L, L)]
                                   * irows_v[b, pl.ds(slc * L, L)])
                        p1 = p1 + (urows_v[b, pl.ds((slc + 1) * L, L)]
                                   * irows_v[b, pl.ds((slc + 1) * L, L)])
                    tmp_v[j, pl.ds(0, L)] = p0 + p1
                accs = [None] * 4
                for t in range(L):
                    v = plsc.load_gather(tmp_v, [iota, jnp.full((L,), t, jnp.int32)])
                    accs[t % 4] = v if accs[t % 4] is None else accs[t % 4] + v
                acc = (accs[0] + accs[1]) + (accs[2] + accs[3])
                acc = acc + ub_v[pl.ds(gbase, L)] + ib_v[pl.ds(gbase, L)] + mean_vec
                out_v[pl.ds(k * CHUNK + gbase, L)] = acc
                return 0

            lax.fori_loop(0, CHUNK // L, group_body, 0)
            if k + 2 < n_chunks:
                fire(k + 2, sl)

        pltpu.sync_copy(out_v, out_h.at[pl.ds(base, n_per_w)])

    return mf_kernel(u_id, i_id, user_emb, user_bias.reshape(-1),
                     item_emb, item_bias.reshape(-1), mean)
